# Initial kernel scaffold; baseline (speedup 1.0000x reference)
import functools

import jax
import jax.numpy as jnp
from jax import lax
from jax.experimental import pallas as pl
from jax.experimental.pallas import tpu as pltpu
from jax.experimental.pallas import tpu_sc as plsc

N = 10000
E = 320000
F = 128
H1 = 32
H2 = 16

NC = 2
NS = 16
NW = NC * NS
EW = E // NW
BLK = 80
NBLK = EW // BLK
K = 5
NGRP = NBLK // K
RPT = N // NS
CD = 8

_mesh = lambda: plsc.VectorSubcoreMesh(core_axis_name="c", subcore_axis_name="s")


@functools.partial(
    pl.kernel,
    out_type=jax.ShapeDtypeStruct((NC, 2, N, CD), jnp.float32),
    mesh=_mesh(),
    scratch_types=[
        pltpu.VMEM_SHARED((N, CD), jnp.float32),
        pltpu.VMEM_SHARED((N, CD), jnp.float32),
        pltpu.VMEM((NBLK, BLK), jnp.int32),
        pltpu.VMEM((NBLK, BLK), jnp.int32),
        pltpu.VMEM((BLK, CD), jnp.float32),
    ],
)
def _degree_kernel(src3, dst3, ones_h, zeros_h, out, acc_o, acc_i,
                   idx_s, idx_d, ones_v):
    c = lax.axis_index("c")
    s = lax.axis_index("s")
    wid = c * NS + s
    lo = s * RPT
    pltpu.sync_copy(zeros_h.at[pl.ds(lo, RPT)], acc_o.at[pl.ds(lo, RPT)])
    pltpu.sync_copy(zeros_h.at[pl.ds(lo, RPT)], acc_i.at[pl.ds(lo, RPT)])
    pltpu.sync_copy(ones_h, ones_v)
    pltpu.sync_copy(src3.at[wid], idx_s)
    pltpu.sync_copy(dst3.at[wid], idx_d)
    plsc.subcore_barrier()

    def body(j, carry):
        pltpu.sync_copy(ones_v, acc_o.at[idx_s.at[j]], add=True)
        pltpu.sync_copy(ones_v, acc_i.at[idx_d.at[j]], add=True)
        return carry

    lax.fori_loop(0, NBLK, body, 0)
    plsc.subcore_barrier()
    pltpu.sync_copy(acc_o.at[pl.ds(lo, RPT)], out.at[c, 0, pl.ds(lo, RPT)])
    pltpu.sync_copy(acc_i.at[pl.ds(lo, RPT)], out.at[c, 1, pl.ds(lo, RPT)])


def _make_prop(C):
    @functools.partial(
        pl.kernel,
        out_type=jax.ShapeDtypeStruct((NC, N, C), jnp.float32),
        mesh=_mesh(),
        scratch_types=[
            pltpu.VMEM_SHARED((N, C), jnp.float32),
            pltpu.VMEM((NBLK, BLK), jnp.int32),
            pltpu.VMEM((NBLK, BLK), jnp.int32),
            pltpu.VMEM((K, BLK, C), jnp.float32),
            pltpu.SemaphoreType.DMA,
        ],
    )
    def prop_k(x_h, src3, dst3, zeros_h, out, acc, idx_s, idx_d, rows, sem):
        c = lax.axis_index("c")
        s = lax.axis_index("s")
        wid = c * NS + s
        lo = s * RPT
        pltpu.sync_copy(zeros_h.at[pl.ds(lo, RPT)], acc.at[pl.ds(lo, RPT)])
        pltpu.sync_copy(src3.at[wid], idx_s)
        pltpu.sync_copy(dst3.at[wid], idx_d)
        plsc.subcore_barrier()

        def body(g, carry):
            cps = [
                pltpu.async_copy(x_h.at[idx_s.at[g * K + t]], rows.at[t], sem)
                for t in range(K)
            ]
            for cp in cps:
                cp.wait()
            for t in range(K):
                pltpu.sync_copy(rows.at[t], acc.at[idx_d.at[g * K + t]],
                                add=True)
            return carry

        lax.fori_loop(0, NGRP, body, 0)
        plsc.subcore_barrier()
        pltpu.sync_copy(acc.at[pl.ds(lo, RPT)], out.at[c, pl.ds(lo, RPT)])

    return prop_k


_prop32 = _make_prop(H1)
_prop16 = _make_prop(H2)


BR = 1000


def _tc1_body(feat_ref, w1_ref, deg_ref, s_ref, x1_ref):
    d = deg_ref[...]
    deg = jnp.maximum(d[:, :, 0] + d[:, :, 1], 1.0)
    sv = lax.rsqrt(deg)
    s_ref[...] = sv
    xw = jnp.dot(feat_ref[...], w1_ref[...], preferred_element_type=jnp.float32)
    x1_ref[...] = xw * sv[:, 0:1]


def _tc1(features, w1, deg_t):
    return pl.pallas_call(
        _tc1_body,
        grid=(N // BR,),
        in_specs=[
            pl.BlockSpec((BR, F), lambda i: (i, 0)),
            pl.BlockSpec((F, H1), lambda i: (0, 0)),
            pl.BlockSpec((BR, 2, NC), lambda i: (i, 0, 0)),
        ],
        out_specs=[
            pl.BlockSpec((BR, 2), lambda i: (i, 0)),
            pl.BlockSpec((BR, H1), lambda i: (i, 0)),
        ],
        out_shape=[
            jax.ShapeDtypeStruct((N, 2), jnp.float32),
            jax.ShapeDtypeStruct((N, H1), jnp.float32),
        ],
    )(features, w1, deg_t)


def _tc2_body(p_ref, s_ref, w2_ref, x2_ref):
    p = p_ref[...]
    sv = s_ref[...]
    h1 = jnp.maximum((p[0] + p[1]) * sv[:, 1:2], 0.0)
    h2 = jnp.dot(h1, w2_ref[...], preferred_element_type=jnp.float32)
    x2_ref[...] = h2 * sv[:, 0:1]


def _tc2(p1, sv, w2):
    return pl.pallas_call(
        _tc2_body,
        grid=(N // BR,),
        in_specs=[
            pl.BlockSpec((NC, BR, H1), lambda i: (0, i, 0)),
            pl.BlockSpec((BR, 2), lambda i: (i, 0)),
            pl.BlockSpec((H1, H2), lambda i: (0, 0)),
        ],
        out_specs=[pl.BlockSpec((BR, H2), lambda i: (i, 0))],
        out_shape=[jax.ShapeDtypeStruct((N, H2), jnp.float32)],
    )(p1, sv, w2)[0]


BM = 1024
BN = 2048


def _tc3_body(pi_ref, si_ref, pj_ref, sj_ref, z_ref, r_ref):
    pi = pi_ref[...]
    zi = (pi[0] + pi[1]) * si_ref[...][:, 1:2]
    pj = pj_ref[...]
    zj = (pj[0] + pj[1]) * sj_ref[...][:, 1:2]
    z_ref[...] = zi
    r_ref[...] = lax.dot_general(
        zi, zj, (((1,), (1,)), ((), ())), preferred_element_type=jnp.float32)


def _tc3(p2, sv):
    gi = pl.cdiv(N, BM)
    gj = pl.cdiv(N, BN)
    return pl.pallas_call(
        _tc3_body,
        grid=(gi, gj),
        in_specs=[
            pl.BlockSpec((NC, BM, H2), lambda i, j: (0, i, 0)),
            pl.BlockSpec((BM, 2), lambda i, j: (i, 0)),
            pl.BlockSpec((NC, BN, H2), lambda i, j: (0, j, 0)),
            pl.BlockSpec((BN, 2), lambda i, j: (j, 0)),
        ],
        out_specs=[
            pl.BlockSpec((BM, H2), lambda i, j: (i, 0)),
            pl.BlockSpec((BM, BN), lambda i, j: (i, j)),
        ],
        out_shape=[
            jax.ShapeDtypeStruct((N, H2), jnp.float32),
            jax.ShapeDtypeStruct((N, N), jnp.float32),
        ],
    )(p2, sv, p2, sv)


def kernel(features, edge_index, W1, W2):
    src3 = edge_index[0].reshape(NW, NBLK, BLK)
    dst3 = edge_index[1].reshape(NW, NBLK, BLK)
    ones8 = jnp.ones((BLK, CD), jnp.float32)
    zeros8 = jnp.zeros((N, CD), jnp.float32)
    zeros32 = jnp.zeros((N, H1), jnp.float32)
    zeros16 = jnp.zeros((N, H2), jnp.float32)

    degp = _degree_kernel(src3, dst3, ones8, zeros8)
    deg_t = jnp.transpose(degp[:, :, :, 0], (2, 1, 0))

    sv, x1 = _tc1(features, W1, deg_t)
    p1 = _prop32(x1, src3, dst3, zeros32)
    x2 = _tc2(p1, sv, W2)
    p2 = _prop16(x2, src3, dst3, zeros16)
    z, recon = _tc3(p2, sv)
    return z, recon


# speedup vs baseline: 13.8034x; 13.8034x over previous
import functools

import jax
import jax.numpy as jnp
from jax import lax
from jax.experimental import pallas as pl
from jax.experimental.pallas import tpu as pltpu
from jax.experimental.pallas import tpu_sc as plsc

N = 10000
E = 320000
F = 128
H1 = 32
H2 = 16

NC = 2
NS = 16
NW = NC * NS
EW = E // NW
BLK = 80
NBLK = EW // BLK
K = 5
NGRP = NBLK // K
NP_ = 10240
RPT = NP_ // NS
CD = 8

_mesh = lambda: plsc.VectorSubcoreMesh(core_axis_name="c", subcore_axis_name="s")


@functools.partial(
    pl.kernel,
    out_type=jax.ShapeDtypeStruct((NC, 2, NP_, CD), jnp.float32),
    mesh=_mesh(),
    scratch_types=[
        pltpu.VMEM_SHARED((NP_, CD), jnp.float32),
        pltpu.VMEM_SHARED((NP_, CD), jnp.float32),
        pltpu.VMEM((NBLK, BLK), jnp.int32),
        pltpu.VMEM((NBLK, BLK), jnp.int32),
        pltpu.VMEM((BLK, CD), jnp.float32),
    ],
    compiler_params=pltpu.CompilerParams(use_tc_tiling_on_sc=False),
)
def _degree_kernel(src3, dst3, ones_h, zeros_h, out, acc_o, acc_i,
                   idx_s, idx_d, ones_v):
    c = lax.axis_index("c")
    s = lax.axis_index("s")
    wid = c * NS + s
    lo = s * RPT
    pltpu.sync_copy(zeros_h.at[pl.ds(lo, RPT)], acc_o.at[pl.ds(lo, RPT)])
    pltpu.sync_copy(zeros_h.at[pl.ds(lo, RPT)], acc_i.at[pl.ds(lo, RPT)])
    pltpu.sync_copy(ones_h, ones_v)
    pltpu.sync_copy(src3.at[wid], idx_s)
    pltpu.sync_copy(dst3.at[wid], idx_d)
    plsc.subcore_barrier()

    def body(j, carry):
        pltpu.sync_copy(ones_v, acc_o.at[idx_s.at[j]], add=True)
        pltpu.sync_copy(ones_v, acc_i.at[idx_d.at[j]], add=True)
        return carry

    lax.fori_loop(0, NBLK, body, 0)
    plsc.subcore_barrier()
    pltpu.sync_copy(acc_o.at[pl.ds(lo, RPT)], out.at[c, 0, pl.ds(lo, RPT)])
    pltpu.sync_copy(acc_i.at[pl.ds(lo, RPT)], out.at[c, 1, pl.ds(lo, RPT)])


def _make_prop(C):
    @functools.partial(
        pl.kernel,
        out_type=jax.ShapeDtypeStruct((NC, NP_, C), jnp.float32),
        mesh=_mesh(),
        scratch_types=[
            pltpu.VMEM_SHARED((NP_, C), jnp.float32),
            pltpu.VMEM((NBLK, BLK), jnp.int32),
            pltpu.VMEM((NBLK, BLK), jnp.int32),
            pltpu.VMEM((K, BLK, C), jnp.float32),
            pltpu.SemaphoreType.DMA,
        ],
        compiler_params=pltpu.CompilerParams(use_tc_tiling_on_sc=False),
    )
    def prop_k(x_h, src3, dst3, zeros_h, out, acc, idx_s, idx_d, rows, sem):
        c = lax.axis_index("c")
        s = lax.axis_index("s")
        wid = c * NS + s
        lo = s * RPT
        pltpu.sync_copy(zeros_h.at[pl.ds(lo, RPT)], acc.at[pl.ds(lo, RPT)])
        pltpu.sync_copy(src3.at[wid], idx_s)
        pltpu.sync_copy(dst3.at[wid], idx_d)
        plsc.subcore_barrier()

        def body(g, carry):
            cps = [
                pltpu.async_copy(x_h.at[idx_s.at[g * K + t]], rows.at[t], sem)
                for t in range(K)
            ]
            for cp in cps:
                cp.wait()
            for t in range(K):
                pltpu.sync_copy(rows.at[t], acc.at[idx_d.at[g * K + t]],
                                add=True)
            return carry

        lax.fori_loop(0, NGRP, body, 0)
        plsc.subcore_barrier()
        pltpu.sync_copy(acc.at[pl.ds(lo, RPT)], out.at[c, pl.ds(lo, RPT)])

    return prop_k


_prop32 = _make_prop(H1)
_prop16 = _make_prop(H2)


BR = 1000


def _tc1_body(feat_ref, w1_ref, deg_ref, s_ref, x1_ref):
    d = deg_ref[...]
    deg = jnp.maximum(d[:, :, 0] + d[:, :, 1], 1.0)
    sv = lax.rsqrt(deg)
    s_ref[...] = sv
    xw = jnp.dot(feat_ref[...], w1_ref[...], preferred_element_type=jnp.float32)
    x1_ref[...] = xw * sv[:, 0:1]


def _tc1(features, w1, deg_t):
    return pl.pallas_call(
        _tc1_body,
        grid=(N // BR,),
        in_specs=[
            pl.BlockSpec((BR, F), lambda i: (i, 0)),
            pl.BlockSpec((F, H1), lambda i: (0, 0)),
            pl.BlockSpec((BR, 2, NC), lambda i: (i, 0, 0)),
        ],
        out_specs=[
            pl.BlockSpec((BR, 2), lambda i: (i, 0)),
            pl.BlockSpec((BR, H1), lambda i: (i, 0)),
        ],
        out_shape=[
            jax.ShapeDtypeStruct((N, 2), jnp.float32),
            jax.ShapeDtypeStruct((N, H1), jnp.float32),
        ],
    )(features, w1, deg_t)


def _tc2_body(p_ref, s_ref, w2_ref, x2_ref):
    p = p_ref[...]
    sv = s_ref[...]
    h1 = jnp.maximum((p[0] + p[1]) * sv[:, 1:2], 0.0)
    h2 = jnp.dot(h1, w2_ref[...], preferred_element_type=jnp.float32)
    x2_ref[...] = h2 * sv[:, 0:1]


def _tc2(p1, sv, w2):
    return pl.pallas_call(
        _tc2_body,
        grid=(N // BR,),
        in_specs=[
            pl.BlockSpec((NC, BR, H1), lambda i: (0, i, 0)),
            pl.BlockSpec((BR, 2), lambda i: (i, 0)),
            pl.BlockSpec((H1, H2), lambda i: (0, 0)),
        ],
        out_specs=[pl.BlockSpec((BR, H2), lambda i: (i, 0))],
        out_shape=[jax.ShapeDtypeStruct((N, H2), jnp.float32)],
    )(p1, sv, w2)[0]


BM = 1024
BN = 2048


def _tc3_body(pi_ref, si_ref, pj_ref, sj_ref, z_ref, r_ref):
    pi = pi_ref[...]
    zi = (pi[0] + pi[1]) * si_ref[...][:, 1:2]
    pj = pj_ref[...]
    zj = (pj[0] + pj[1]) * sj_ref[...][:, 1:2]
    z_ref[...] = zi
    r_ref[...] = lax.dot_general(
        zi, zj, (((1,), (1,)), ((), ())), preferred_element_type=jnp.float32)


def _tc3(p2, sv):
    gi = pl.cdiv(N, BM)
    gj = pl.cdiv(N, BN)
    return pl.pallas_call(
        _tc3_body,
        grid=(gi, gj),
        in_specs=[
            pl.BlockSpec((NC, BM, H2), lambda i, j: (0, i, 0)),
            pl.BlockSpec((BM, 2), lambda i, j: (i, 0)),
            pl.BlockSpec((NC, BN, H2), lambda i, j: (0, j, 0)),
            pl.BlockSpec((BN, 2), lambda i, j: (j, 0)),
        ],
        out_specs=[
            pl.BlockSpec((BM, H2), lambda i, j: (i, 0)),
            pl.BlockSpec((BM, BN), lambda i, j: (i, j)),
        ],
        out_shape=[
            jax.ShapeDtypeStruct((N, H2), jnp.float32),
            jax.ShapeDtypeStruct((N, N), jnp.float32),
        ],
    )(p2, sv, p2, sv)


def kernel(features, edge_index, W1, W2):
    src3 = edge_index[0].reshape(NW, NBLK, BLK)
    dst3 = edge_index[1].reshape(NW, NBLK, BLK)
    ones8 = jnp.ones((BLK, CD), jnp.float32)
    zeros8 = jnp.zeros((NP_, CD), jnp.float32)
    zeros32 = jnp.zeros((NP_, H1), jnp.float32)
    zeros16 = jnp.zeros((NP_, H2), jnp.float32)

    degp = _degree_kernel(src3, dst3, ones8, zeros8)
    deg_t = jnp.transpose(degp[:, :, :N, 0], (2, 1, 0))

    sv, x1 = _tc1(features, W1, deg_t)
    p1 = _prop32(x1, src3, dst3, zeros32)
    x2 = _tc2(p1, sv, W2)
    p2 = _prop16(x2, src3, dst3, zeros16)
    z, recon = _tc3(p2, sv)
    return z, recon


# speedup vs baseline: 23.2221x; 1.6823x over previous
import functools

import jax
import jax.numpy as jnp
from jax import lax
from jax.experimental import pallas as pl
from jax.experimental.pallas import tpu as pltpu
from jax.experimental.pallas import tpu_sc as plsc

N = 10000
E = 320000
F = 128
H1 = 32
H2 = 16

NC = 2
NS = 16
NW = NC * NS
BLK = 80
NBLK = 125
K = 5
NGRP = NBLK // K
NGRP2 = (NGRP - 1) // 2
NP_ = 10240
RPT = NP_ // NS
CD = 8
XPT = 624

_mesh = lambda: plsc.VectorSubcoreMesh(core_axis_name="c", subcore_axis_name="s")


@functools.partial(
    pl.kernel,
    out_type=jax.ShapeDtypeStruct((NC, 2, NP_, CD), jnp.float32),
    mesh=_mesh(),
    scratch_types=[
        pltpu.VMEM_SHARED((NP_, CD), jnp.float32),
        pltpu.VMEM_SHARED((NP_, CD), jnp.float32),
        pltpu.VMEM((NBLK, BLK), jnp.int32),
        pltpu.VMEM((NBLK, BLK), jnp.int32),
        pltpu.VMEM((BLK, CD), jnp.float32),
        pltpu.SemaphoreType.DMA,
    ],
    compiler_params=pltpu.CompilerParams(use_tc_tiling_on_sc=False),
)
def _degree_kernel(src3, dst3, ones_h, zeros_h, out, acc_o, acc_i,
                   idx_s, idx_d, ones_v, sem):
    c = lax.axis_index("c")
    s = lax.axis_index("s")
    wid = c * NS + s
    lo = s * RPT
    pltpu.sync_copy(zeros_h.at[pl.ds(lo, RPT)], acc_o.at[pl.ds(lo, RPT)])
    pltpu.sync_copy(zeros_h.at[pl.ds(lo, RPT)], acc_i.at[pl.ds(lo, RPT)])
    pltpu.sync_copy(ones_h, ones_v)
    pltpu.sync_copy(src3.at[wid], idx_s)
    pltpu.sync_copy(dst3.at[wid], idx_d)
    plsc.subcore_barrier()

    def body(j, carry):
        pltpu.async_copy(ones_v, acc_o.at[idx_s.at[j]], sem, add=True)
        pltpu.async_copy(ones_v, acc_i.at[idx_d.at[j]], sem, add=True)

        @pl.when(j > 0)
        def _():
            pltpu.make_async_copy(ones_v, acc_o.at[idx_s.at[j]], sem).wait()
            pltpu.make_async_copy(ones_v, acc_i.at[idx_d.at[j]], sem).wait()

        return carry

    lax.fori_loop(0, NBLK, body, 0)
    pltpu.make_async_copy(ones_v, acc_o.at[idx_s.at[0]], sem).wait()
    pltpu.make_async_copy(ones_v, acc_i.at[idx_d.at[0]], sem).wait()
    plsc.subcore_barrier()
    pltpu.sync_copy(acc_o.at[pl.ds(lo, RPT)], out.at[c, 0, pl.ds(lo, RPT)])
    pltpu.sync_copy(acc_i.at[pl.ds(lo, RPT)], out.at[c, 1, pl.ds(lo, RPT)])


def _make_prop(C):
    @functools.partial(
        pl.kernel,
        out_type=jax.ShapeDtypeStruct((NC, NP_, C), jnp.float32),
        mesh=_mesh(),
        scratch_types=[
            pltpu.VMEM_SHARED((NP_, C), jnp.float32),
            pltpu.VMEM_SHARED((N, C), jnp.float32),
            pltpu.VMEM((NBLK, BLK), jnp.int32),
            pltpu.VMEM((NBLK, BLK), jnp.int32),
            pltpu.VMEM((K, BLK, C), jnp.float32),
            pltpu.VMEM((K, BLK, C), jnp.float32),
            pltpu.SemaphoreType.DMA,
            pltpu.SemaphoreType.DMA,
        ],
        compiler_params=pltpu.CompilerParams(use_tc_tiling_on_sc=False),
    )
    def prop_k(x_h, src3, dst3, zeros_h, out, acc, xs, idx_s, idx_d,
               rows_a, rows_b, sem_a, sem_b):
        c = lax.axis_index("c")
        s = lax.axis_index("s")
        wid = c * NS + s
        lo = s * RPT
        pltpu.sync_copy(zeros_h.at[pl.ds(lo, RPT)], acc.at[pl.ds(lo, RPT)])
        pltpu.sync_copy(src3.at[wid], idx_s)
        pltpu.sync_copy(dst3.at[wid], idx_d)
        pltpu.sync_copy(x_h.at[pl.ds(s * XPT, XPT)], xs.at[pl.ds(s * XPT, XPT)])

        @pl.when(s == 0)
        def _():
            pltpu.sync_copy(x_h.at[pl.ds(NS * XPT, N - NS * XPT)],
                            xs.at[pl.ds(NS * XPT, N - NS * XPT)])

        plsc.subcore_barrier()

        def fire(g, rows, sem):
            for t in range(K):
                pltpu.async_copy(xs.at[idx_s.at[g * K + t]], rows.at[t], sem)

        def drain(g, rows, sem):
            for t in range(K):
                pltpu.make_async_copy(
                    x_h.at[idx_s.at[g * K + t]], rows.at[t], sem).wait()

        def scatter(g, rows):
            for t in range(K):
                pltpu.sync_copy(rows.at[t], acc.at[idx_d.at[g * K + t]],
                                add=True)

        fire(0, rows_a, sem_a)

        def body(h, carry):
            g = 2 * h
            fire(g + 1, rows_b, sem_b)
            drain(g, rows_a, sem_a)
            scatter(g, rows_a)
            fire(g + 2, rows_a, sem_a)
            drain(g + 1, rows_b, sem_b)
            scatter(g + 1, rows_b)
            return carry

        lax.fori_loop(0, NGRP2, body, 0)
        g_last = 2 * NGRP2
        drain(g_last, rows_a, sem_a)
        scatter(g_last, rows_a)
        plsc.subcore_barrier()
        pltpu.sync_copy(acc.at[pl.ds(lo, RPT)], out.at[c, pl.ds(lo, RPT)])

    return prop_k


_prop32 = _make_prop(H1)
_prop16 = _make_prop(H2)


BR = N


def _tc1a_body(feat_ref, w1_ref, xw_ref):
    xw_ref[...] = jnp.dot(feat_ref[...], w1_ref[...],
                          preferred_element_type=jnp.float32)


def _tc1a(features, w1):
    return pl.pallas_call(
        _tc1a_body,
        grid=(1,),
        in_specs=[
            pl.BlockSpec((BR, F), lambda i: (0, 0)),
            pl.BlockSpec((F, H1), lambda i: (0, 0)),
        ],
        out_specs=pl.BlockSpec((BR, H1), lambda i: (0, 0)),
        out_shape=jax.ShapeDtypeStruct((N, H1), jnp.float32),
    )(features, w1)


def _tc1b_body(deg_ref, xw_ref, s_ref, x1_ref):
    d = deg_ref[...]
    d_o = d[0, 0, :, 0:1] + d[1, 0, :, 0:1]
    d_i = d[0, 1, :, 0:1] + d[1, 1, :, 0:1]
    sv = jnp.concatenate(
        [lax.rsqrt(jnp.maximum(d_o, 1.0)), lax.rsqrt(jnp.maximum(d_i, 1.0))],
        axis=1)
    s_ref[...] = sv
    x1_ref[...] = xw_ref[...] * sv[:, 0:1]


def _tc1b(degp, xw):
    return pl.pallas_call(
        _tc1b_body,
        grid=(1,),
        in_specs=[
            pl.BlockSpec((NC, 2, BR, CD), lambda i: (0, 0, 0, 0)),
            pl.BlockSpec((BR, H1), lambda i: (0, 0)),
        ],
        out_specs=[
            pl.BlockSpec((BR, 2), lambda i: (0, 0)),
            pl.BlockSpec((BR, H1), lambda i: (0, 0)),
        ],
        out_shape=[
            jax.ShapeDtypeStruct((N, 2), jnp.float32),
            jax.ShapeDtypeStruct((N, H1), jnp.float32),
        ],
    )(degp, xw)


def _tc2_body(p_ref, s_ref, w2_ref, x2_ref):
    p = p_ref[...]
    sv = s_ref[...]
    h1 = jnp.maximum((p[0] + p[1]) * sv[:, 1:2], 0.0)
    h2 = jnp.dot(h1, w2_ref[...], preferred_element_type=jnp.float32)
    x2_ref[...] = h2 * sv[:, 0:1]


def _tc2(p1, sv, w2):
    return pl.pallas_call(
        _tc2_body,
        grid=(1,),
        in_specs=[
            pl.BlockSpec((NC, BR, H1), lambda i: (0, 0, 0)),
            pl.BlockSpec((BR, 2), lambda i: (0, 0)),
            pl.BlockSpec((H1, H2), lambda i: (0, 0)),
        ],
        out_specs=[pl.BlockSpec((BR, H2), lambda i: (0, 0))],
        out_shape=[jax.ShapeDtypeStruct((N, H2), jnp.float32)],
    )(p1, sv, w2)[0]


BM = 2048
BN = 2048


def _tc3_body(pi_ref, si_ref, pj_ref, sj_ref, z_ref, r_ref):
    pi = pi_ref[...]
    zi = (pi[0] + pi[1]) * si_ref[...][:, 1:2]
    pj = pj_ref[...]
    zj = (pj[0] + pj[1]) * sj_ref[...][:, 1:2]
    z_ref[...] = zi
    r_ref[...] = lax.dot_general(
        zi, zj, (((1,), (1,)), ((), ())), preferred_element_type=jnp.float32)


def _tc3(p2, sv):
    gi = pl.cdiv(N, BM)
    gj = pl.cdiv(N, BN)
    return pl.pallas_call(
        _tc3_body,
        grid=(gi, gj),
        in_specs=[
            pl.BlockSpec((NC, BM, H2), lambda i, j: (0, i, 0)),
            pl.BlockSpec((BM, 2), lambda i, j: (i, 0)),
            pl.BlockSpec((NC, BN, H2), lambda i, j: (0, j, 0)),
            pl.BlockSpec((BN, 2), lambda i, j: (j, 0)),
        ],
        out_specs=[
            pl.BlockSpec((BM, H2), lambda i, j: (i, 0)),
            pl.BlockSpec((BM, BN), lambda i, j: (i, j)),
        ],
        out_shape=[
            jax.ShapeDtypeStruct((N, H2), jnp.float32),
            jax.ShapeDtypeStruct((N, N), jnp.float32),
        ],
    )(p2, sv, p2, sv)


def kernel(features, edge_index, W1, W2):
    src3 = edge_index[0].reshape(NW, NBLK, BLK)
    dst3 = edge_index[1].reshape(NW, NBLK, BLK)
    ones8 = jnp.ones((BLK, CD), jnp.float32)
    zeros8 = jnp.zeros((NP_, CD), jnp.float32)
    zeros32 = jnp.zeros((NP_, H1), jnp.float32)
    zeros16 = jnp.zeros((NP_, H2), jnp.float32)

    degp = _degree_kernel(src3, dst3, ones8, zeros8)
    xw = _tc1a(features, W1)
    sv, x1 = _tc1b(degp, xw)
    p1 = _prop32(x1, src3, dst3, zeros32)
    x2 = _tc2(p1, sv, W2)
    p2 = _prop16(x2, src3, dst3, zeros16)
    z, recon = _tc3(p2, sv)
    return z, recon
